# baseline (speedup 1.0000x reference)
import functools

import jax
import jax.numpy as jnp
from jax import lax
from jax.experimental import pallas as pl
from jax.experimental.pallas import tpu as pltpu
from jax.experimental.pallas import tpu_sc as plsc

N_FIELDS = 26
VOCAB_P1 = 100001
EMBED_DIM = 64
BATCH = 4096

NUM_CORES = 2
NUM_SUBCORES = 16
NW = NUM_CORES * NUM_SUBCORES

ROWS = BATCH * N_FIELDS
BPW = ROWS // NW
CHUNK = 128
NCHUNK = BPW // CHUNK


@functools.partial(
    pl.kernel,
    out_type=jax.ShapeDtypeStruct((ROWS, EMBED_DIM), jnp.float32),
    mesh=plsc.VectorSubcoreMesh(core_axis_name="c", subcore_axis_name="s"),
    scratch_types=[
        pltpu.VMEM((NCHUNK, CHUNK), jnp.int32),
        pltpu.VMEM((CHUNK, EMBED_DIM), jnp.float32),
        pltpu.VMEM((CHUNK, EMBED_DIM), jnp.float32),
        pltpu.SemaphoreType.DMA,
        pltpu.SemaphoreType.DMA,
    ],
    compiler_params=pltpu.CompilerParams(use_tc_tiling_on_sc=False),
)
def _gather(tab_hbm, idx_hbm, out_hbm, idx_v, buf0, buf1, sem0, sem1):
    wid = lax.axis_index("s") * NUM_CORES + lax.axis_index("c")
    base = wid * BPW
    pltpu.sync_copy(idx_hbm.at[wid], idx_v)

    bufs = (buf0, buf1)
    sems = (sem0, sem1)
    handles = [None, None]
    handles[0] = pltpu.async_copy(tab_hbm.at[idx_v.at[0]], buf0, sem0)
    for j in range(NCHUNK):
        cur = j % 2
        if j + 1 < NCHUNK:
            nxt = (j + 1) % 2
            handles[nxt] = pltpu.async_copy(
                tab_hbm.at[idx_v.at[j + 1]], bufs[nxt], sems[nxt])
        handles[cur].wait()
        pltpu.sync_copy(bufs[cur], out_hbm.at[pl.ds(base + j * CHUNK, CHUNK)])


def kernel(x, tables):
    offs = jnp.arange(N_FIELDS, dtype=jnp.int32) * VOCAB_P1
    idx = (x.astype(jnp.int32) + offs[None, :]).reshape(NW, NCHUNK, CHUNK)
    tab = tables.reshape(N_FIELDS * VOCAB_P1, EMBED_DIM)
    out = _gather(tab, idx)
    return out.reshape(BATCH, N_FIELDS, EMBED_DIM)


# speedup vs baseline: 8.3534x; 8.3534x over previous
import functools

import jax
import jax.numpy as jnp
from jax import lax
from jax.experimental import pallas as pl
from jax.experimental.pallas import tpu as pltpu
from jax.experimental.pallas import tpu_sc as plsc

N_FIELDS = 26
VOCAB_P1 = 100001
EMBED_DIM = 64
BATCH = 4096

NUM_CORES = 2
NUM_SUBCORES = 16
NW = NUM_CORES * NUM_SUBCORES

CHUNK = BATCH // NW
BPW = N_FIELDS * CHUNK


@functools.partial(
    pl.kernel,
    out_type=jax.ShapeDtypeStruct((N_FIELDS, BATCH, EMBED_DIM), jnp.float32),
    mesh=plsc.VectorSubcoreMesh(core_axis_name="c", subcore_axis_name="s"),
    scratch_types=[
        pltpu.VMEM((BPW,), jnp.int32),
        pltpu.VMEM_SHARED((NUM_SUBCORES, BPW), jnp.int32),
        pltpu.SMEM((2, CHUNK), jnp.int32),
        pltpu.VMEM((CHUNK, EMBED_DIM), jnp.float32),
        pltpu.VMEM((CHUNK, EMBED_DIM), jnp.float32),
        pltpu.SemaphoreType.DMA,
        pltpu.SemaphoreType.DMA,
        pltpu.SemaphoreType.DMA,
    ],
)
def _gather(tab_hbm, idx_hbm, out_hbm, idx_v, idx_sp, idx_s, buf0, buf1,
            sem0, sem1, sem_i):
    sid = lax.axis_index("s")
    wid = sid * NUM_CORES + lax.axis_index("c")
    base = wid * CHUNK

    bufs = (buf0, buf1)
    sems = (sem0, sem1)

    pltpu.sync_copy(idx_hbm.at[pl.ds(wid * BPW, BPW)], idx_v)
    pltpu.sync_copy(idx_v, idx_sp.at[sid])

    def fire(f, p):
        pltpu.async_copy(idx_sp.at[sid, pl.ds(f * CHUNK, CHUNK)],
                         idx_s.at[p], sem_i).wait()
        buf = bufs[p]

        def row(i):
            r = idx_s[p, i]
            pltpu.async_copy(tab_hbm.at[f].at[pl.ds(r, 1)],
                             buf.at[pl.ds(i, 1)], sems[p])
        pl.loop(0, CHUNK)(row)

    def drain_and_store(f, p):
        pltpu.make_async_copy(
            out_hbm.at[f].at[pl.ds(base, CHUNK)], bufs[p], sems[p]).wait()
        pltpu.sync_copy(bufs[p], out_hbm.at[f].at[pl.ds(base, CHUNK)])

    fire(0, 0)
    for f in range(1, N_FIELDS):
        fire(f, f % 2)
        drain_and_store(f - 1, (f - 1) % 2)
    drain_and_store(N_FIELDS - 1, (N_FIELDS - 1) % 2)


def kernel(x, tables):
    idx = (x.astype(jnp.int32)
           .reshape(NW, CHUNK, N_FIELDS)
           .transpose(0, 2, 1)
           .reshape(NW * BPW))
    out = _gather(tables, idx)
    return out.transpose(1, 0, 2)
